# Initial kernel scaffold; baseline (speedup 1.0000x reference)
import jax
import jax.numpy as jnp
from jax.experimental import pallas as pl


def kernel(x, edge_index, Wq, bq, Wk, bk, Wv, bv, Ws, bs):
    raise NotImplementedError("write your pallas kernel here")


# speedup vs baseline: 1.8046x; 1.8046x over previous
import dataclasses
import functools

import jax
import jax.numpy as jnp
import numpy as np
from jax import lax
from jax.experimental import pallas as pl
from jax.experimental.pallas import tpu as pltpu
from jax.experimental.pallas import tpu_sc as plsc

N = 10000
E = 320000
D = 128
NC = 2
NS = 16
NW = NC * NS
EPW = E // NW
C = 80
NCHUNK = EPW // C
ROW_CHUNKS = N // C
PK = 16
DROWS = 640
NPAD = DROWS * PK
DCHUNK = DROWS // NS


def _proj_body(x_ref, wq_ref, bq_ref, wk_ref, bk_ref, wv_ref, bv_ref,
               ws_ref, bs_ref, q_ref, k_ref, v_ref, s_ref):
    x = x_ref[...]
    dn = (((1,), (1,)), ((), ()))
    hi = lax.Precision.HIGHEST
    q_ref[...] = lax.dot_general(x, wq_ref[...], dn, precision=hi) + bq_ref[...]
    k_ref[...] = lax.dot_general(x, wk_ref[...], dn, precision=hi) + bk_ref[...]
    v_ref[...] = lax.dot_general(x, wv_ref[...], dn, precision=hi) + bv_ref[...]
    s_ref[...] = lax.dot_general(x, ws_ref[...], dn, precision=hi) + bs_ref[...]


def _proj(x, Wq, bq, Wk, bk, Wv, bv, Ws, bs):
    R = 2000
    row_spec = pl.BlockSpec((R, D), lambda i: (i, 0))
    w_spec = pl.BlockSpec((D, D), lambda i: (0, 0))
    b_spec = pl.BlockSpec((D,), lambda i: (0,))
    out = jax.ShapeDtypeStruct((N, D), jnp.float32)
    out_pad = jax.ShapeDtypeStruct((NPAD, D), jnp.float32)
    return pl.pallas_call(
        _proj_body,
        grid=(N // R,),
        in_specs=[row_spec] + [w_spec, b_spec] * 4,
        out_specs=[row_spec] * 4,
        out_shape=[out, out, out, out_pad],
    )(x, Wq, bq, Wk, bk, Wv, bv, Ws, bs)


def _edge_agg(q, k, v, src, dst):
    mesh = plsc.VectorSubcoreMesh(core_axis_name="c", subcore_axis_name="s")
    inv_sqrt_d = np.float32(1.0 / np.sqrt(D))
    cp = pltpu.CompilerParams()
    if "needs_layout_passes" in pltpu.CompilerParams.__dataclass_fields__:
        cp = dataclasses.replace(cp, needs_layout_passes=False)

    @functools.partial(
        pl.kernel,
        out_type=(
            jax.ShapeDtypeStruct((NC, NPAD, D), jnp.float32),
            jax.ShapeDtypeStruct((NC, NPAD), jnp.float32),
        ),
        mesh=mesh,
        compiler_params=cp,
        scratch_types=[
            pltpu.VMEM((C,), jnp.int32),
            pltpu.VMEM((C,), jnp.int32),
            pltpu.VMEM((C,), jnp.int32),
            pltpu.VMEM((C, D), jnp.float32),
            pltpu.VMEM((C, D), jnp.float32),
            pltpu.VMEM((C, D), jnp.float32),
            pltpu.VMEM((C, D), jnp.float32),
            pltpu.VMEM((DCHUNK * PK,), jnp.float32),
            pltpu.VMEM_SHARED((N, D), jnp.float32),
            pltpu.VMEM_SHARED((DROWS, D), jnp.float32),
        ],
    )
    def agg_kernel(q_hbm, k_hbm, v_hbm, src_hbm, dst_hbm, agg_hbm, den_hbm,
                   sidx, didx, didxp, qg, kg, msg, denmsg, den1,
                   acc, dent):
        cid = lax.axis_index("c")
        sid = lax.axis_index("s")
        wid = cid * NS + sid
        zv = jnp.zeros((16,), jnp.float32)
        lane = jnp.arange(16, dtype=jnp.int32)

        @pl.loop(0, C)
        def _(r):
            for j in range(D // 16):
                msg[r, pl.ds(j * 16, 16)] = zv
                denmsg[r, pl.ds(j * 16, 16)] = zv

        @pl.loop(0, (ROW_CHUNKS + NS - 1) // NS)
        def _(t):
            rc = sid + t * NS

            @pl.when(rc < ROW_CHUNKS)
            def _():
                pltpu.sync_copy(msg, acc.at[pl.ds(rc * C, C)])

        pltpu.sync_copy(msg.at[pl.ds(0, DCHUNK)],
                        dent.at[pl.ds(sid * DCHUNK, DCHUNK)])
        plsc.subcore_barrier()

        @pl.loop(0, NCHUNK)
        def _(i):
            e0 = wid * EPW + i * C
            pltpu.sync_copy(src_hbm.at[pl.ds(e0, C)], sidx)
            pltpu.sync_copy(dst_hbm.at[pl.ds(e0, C)], didx)
            pltpu.sync_copy(q_hbm.at[didx], qg)
            pltpu.sync_copy(k_hbm.at[sidx], kg)
            pltpu.sync_copy(v_hbm.at[sidx], msg)

            for e0v in range(0, C, 16):
                eids = lane + e0v

                def dot_body(f, s):
                    fids = jnp.full((16,), f, jnp.int32)
                    qc = plsc.load_gather(qg, [eids, fids])
                    kc = plsc.load_gather(kg, [eids, fids])
                    return s + qc * kc

                alpha = lax.fori_loop(0, D, dot_body,
                                      jnp.zeros((16,), jnp.float32),
                                      unroll=4)
                wv = jnp.exp(alpha * inv_sqrt_d)

                dv = didx[pl.ds(e0v, 16)]
                didxp[pl.ds(e0v, 16)] = lax.shift_right_logical(dv, 4)
                cols = lax.shift_left(lax.bitwise_and(dv, 15), 3)
                plsc.store_scatter(denmsg, [eids, cols], wv)

                @pl.loop(0, D)
                def _(f):
                    fids = jnp.full((16,), f, jnp.int32)
                    vc = plsc.load_gather(msg, [eids, fids])
                    plsc.store_scatter(msg, [eids, fids], vc * wv)

            pltpu.sync_copy(msg, acc.at[didx], add=True)
            pltpu.sync_copy(denmsg, dent.at[didxp], add=True)

            for e0v in range(0, C, 16):
                dv = didx[pl.ds(e0v, 16)]
                cols = lax.shift_left(lax.bitwise_and(dv, 15), 3)
                plsc.store_scatter(denmsg, [lane + e0v, cols], zv)

        plsc.subcore_barrier()

        @pl.loop(0, (ROW_CHUNKS + NS - 1) // NS)
        def _(t):
            rc = sid + t * NS

            @pl.when(rc < ROW_CHUNKS)
            def _():
                pltpu.sync_copy(acc.at[pl.ds(rc * C, C)],
                                agg_hbm.at[cid, pl.ds(rc * C, C)])

        pltpu.sync_copy(dent.at[pl.ds(sid * DCHUNK, DCHUNK)],
                        denmsg.at[pl.ds(0, DCHUNK)])

        @pl.loop(0, DCHUNK * PK // 16)
        def _(g):
            n = g * 16 + lane
            rows = lax.shift_right_logical(n, 4)
            cols = lax.shift_left(lax.bitwise_and(n, 15), 3)
            den1[pl.ds(g * 16, 16)] = plsc.load_gather(denmsg, [rows, cols])

        pltpu.sync_copy(den1, den_hbm.at[cid, pl.ds(sid * DCHUNK * PK,
                                                    DCHUNK * PK)])

    return agg_kernel(q, k, v, src, dst)


def _finish_body(acc_ref, den_ref, skip_ref, out_ref):
    agg = acc_ref[0] + acc_ref[1]
    den = den_ref[0] + den_ref[1]
    out_ref[...] = jnp.maximum(agg / (den[:, None] + 1e-16) + skip_ref[...],
                               0.0)


def _finish(acc2, den2, skip):
    R = 2048
    return pl.pallas_call(
        _finish_body,
        grid=(NPAD // R,),
        in_specs=[
            pl.BlockSpec((NC, R, D), lambda i: (0, i, 0)),
            pl.BlockSpec((NC, R), lambda i: (0, i)),
            pl.BlockSpec((R, D), lambda i: (i, 0)),
        ],
        out_specs=pl.BlockSpec((R, D), lambda i: (i, 0)),
        out_shape=jax.ShapeDtypeStruct((NPAD, D), jnp.float32),
    )(acc2, den2, skip)


def kernel(x, edge_index, Wq, bq, Wk, bk, Wv, bv, Ws, bs):
    src = edge_index[0]
    dst = edge_index[1]
    q, k, v, skip = _proj(x, Wq, bq, Wk, bk, Wv, bv, Ws, bs)
    acc2, den2 = _edge_agg(q, k, v, src, dst)
    return _finish(acc2, den2, skip)[:N]


# speedup vs baseline: 5.9338x; 3.2881x over previous
import dataclasses
import functools

import jax
import jax.numpy as jnp
import numpy as np
from jax import lax
from jax.experimental import pallas as pl
from jax.experimental.pallas import tpu as pltpu
from jax.experimental.pallas import tpu_sc as plsc

N = 10000
E = 320000
D = 128
NC = 2
NS = 16
NW = NC * NS
EPW = E // NW
C = 80
NCHUNK = EPW // C
ROW_CHUNKS = N // C
PK = 16
DROWS = 640
NPAD = DROWS * PK
DCHUNK = DROWS // NS


def _proj_body(x_ref, wq_ref, bq_ref, wk_ref, bk_ref, wv_ref, bv_ref,
               ws_ref, bs_ref, q_ref, k_ref, v_ref, s_ref):
    x = x_ref[...]
    dn = (((1,), (1,)), ((), ()))
    hi = lax.Precision.HIGHEST
    q_ref[...] = lax.dot_general(x, wq_ref[...], dn, precision=hi) + bq_ref[...]
    k_ref[...] = lax.dot_general(x, wk_ref[...], dn, precision=hi) + bk_ref[...]
    v_ref[...] = lax.dot_general(x, wv_ref[...], dn, precision=hi) + bv_ref[...]
    s_ref[...] = lax.dot_general(x, ws_ref[...], dn, precision=hi) + bs_ref[...]


def _proj(x, Wq, bq, Wk, bk, Wv, bv, Ws, bs):
    R = 2000
    row_spec = pl.BlockSpec((R, D), lambda i: (i, 0))
    w_spec = pl.BlockSpec((D, D), lambda i: (0, 0))
    b_spec = pl.BlockSpec((D,), lambda i: (0,))
    out = jax.ShapeDtypeStruct((N, D), jnp.float32)
    out_pad = jax.ShapeDtypeStruct((NPAD, D), jnp.float32)
    return pl.pallas_call(
        _proj_body,
        grid=(N // R,),
        in_specs=[row_spec] + [w_spec, b_spec] * 4,
        out_specs=[row_spec] * 4,
        out_shape=[out, out, out, out_pad],
    )(x, Wq, bq, Wk, bk, Wv, bv, Ws, bs)


def _edge_agg(q, k, v, src, dst):
    mesh = plsc.VectorSubcoreMesh(core_axis_name="c", subcore_axis_name="s")
    inv_sqrt_d = np.float32(1.0 / np.sqrt(D))
    cp = pltpu.CompilerParams()
    if "needs_layout_passes" in pltpu.CompilerParams.__dataclass_fields__:
        cp = dataclasses.replace(cp, needs_layout_passes=False)

    @functools.partial(
        pl.kernel,
        out_type=(
            jax.ShapeDtypeStruct((NC, NPAD, D), jnp.float32),
            jax.ShapeDtypeStruct((NC, NPAD), jnp.float32),
        ),
        mesh=mesh,
        compiler_params=cp,
        scratch_types=[
            pltpu.VMEM((C,), jnp.int32),
            pltpu.VMEM((C,), jnp.int32),
            pltpu.VMEM((C,), jnp.int32),
            pltpu.VMEM((C, D), jnp.float32),
            pltpu.VMEM((C, D), jnp.float32),
            pltpu.VMEM((C, D), jnp.float32),
            pltpu.VMEM((C, D), jnp.float32),
            pltpu.VMEM((C,), jnp.float32),
            pltpu.VMEM((DCHUNK * PK,), jnp.float32),
            pltpu.VMEM_SHARED((N, D), jnp.float32),
            pltpu.VMEM_SHARED((DROWS, D), jnp.float32),
        ],
    )
    def agg_kernel(q_hbm, k_hbm, v_hbm, src_hbm, dst_hbm, agg_hbm, den_hbm,
                   sidx, didx, didxp, qg, kg, msg, denmsg, abuf, den1,
                   acc, dent):
        cid = lax.axis_index("c")
        sid = lax.axis_index("s")
        wid = cid * NS + sid
        zv = jnp.zeros((16,), jnp.float32)
        lane = jnp.arange(16, dtype=jnp.int32)
        mask15 = lane == 15

        @pl.loop(0, C)
        def _(r):
            for j in range(D // 16):
                msg[r, pl.ds(j * 16, 16)] = zv
                denmsg[r, pl.ds(j * 16, 16)] = zv

        @pl.loop(0, (ROW_CHUNKS + NS - 1) // NS)
        def _(t):
            rc = sid + t * NS

            @pl.when(rc < ROW_CHUNKS)
            def _():
                pltpu.sync_copy(msg, acc.at[pl.ds(rc * C, C)])

        pltpu.sync_copy(msg.at[pl.ds(0, DCHUNK)],
                        dent.at[pl.ds(sid * DCHUNK, DCHUNK)])
        plsc.subcore_barrier()

        @pl.loop(0, NCHUNK)
        def _(i):
            e0 = wid * EPW + i * C
            pltpu.sync_copy(src_hbm.at[pl.ds(e0, C)], sidx)
            pltpu.sync_copy(dst_hbm.at[pl.ds(e0, C)], didx)
            pltpu.sync_copy(q_hbm.at[didx], qg)
            pltpu.sync_copy(k_hbm.at[sidx], kg)
            pltpu.sync_copy(v_hbm.at[sidx], msg)

            @pl.loop(0, C, step=4)
            def _(e0):
                for u in range(4):
                    e = e0 + u
                    p = [qg[e, pl.ds(16 * j, 16)] * kg[e, pl.ds(16 * j, 16)]
                         for j in range(8)]
                    t = ((p[0] + p[1]) + (p[2] + p[3])) + \
                        ((p[4] + p[5]) + (p[6] + p[7]))
                    c = plsc.cumsum(t)
                    wv = jnp.exp(c * inv_sqrt_d)
                    plsc.store_scatter(abuf, [jnp.full((16,), e, jnp.int32)],
                                       wv, mask=mask15)

            for e0v in range(0, C, 16):
                wv16 = abuf[pl.ds(e0v, 16)]
                dv = didx[pl.ds(e0v, 16)]
                didxp[pl.ds(e0v, 16)] = lax.shift_right_logical(dv, 4)
                cols = lax.shift_left(lax.bitwise_and(dv, 15), 3)
                plsc.store_scatter(denmsg, [lane + e0v, cols], wv16)

            @pl.loop(0, C // 16)
            def _(g):
                e0g = g * 16
                wv16 = abuf[pl.ds(e0g, 16)]
                for u in range(16):
                    w = wv16[u]
                    for j in range(8):
                        msg[e0g + u, pl.ds(16 * j, 16)] = \
                            msg[e0g + u, pl.ds(16 * j, 16)] * w

            pltpu.sync_copy(msg, acc.at[didx], add=True)
            pltpu.sync_copy(denmsg, dent.at[didxp], add=True)

            for e0v in range(0, C, 16):
                dv = didx[pl.ds(e0v, 16)]
                cols = lax.shift_left(lax.bitwise_and(dv, 15), 3)
                plsc.store_scatter(denmsg, [lane + e0v, cols], zv)

        plsc.subcore_barrier()

        @pl.loop(0, (ROW_CHUNKS + NS - 1) // NS)
        def _(t):
            rc = sid + t * NS

            @pl.when(rc < ROW_CHUNKS)
            def _():
                pltpu.sync_copy(acc.at[pl.ds(rc * C, C)],
                                agg_hbm.at[cid, pl.ds(rc * C, C)])

        pltpu.sync_copy(dent.at[pl.ds(sid * DCHUNK, DCHUNK)],
                        denmsg.at[pl.ds(0, DCHUNK)])

        @pl.loop(0, DCHUNK * PK // 16)
        def _(g):
            n = g * 16 + lane
            rows = lax.shift_right_logical(n, 4)
            cols = lax.shift_left(lax.bitwise_and(n, 15), 3)
            den1[pl.ds(g * 16, 16)] = plsc.load_gather(denmsg, [rows, cols])

        pltpu.sync_copy(den1, den_hbm.at[cid, pl.ds(sid * DCHUNK * PK,
                                                    DCHUNK * PK)])

    return agg_kernel(q, k, v, src, dst)


def _finish_body(acc_ref, den_ref, skip_ref, out_ref):
    agg = acc_ref[0] + acc_ref[1]
    den = den_ref[0] + den_ref[1]
    out_ref[...] = jnp.maximum(agg / (den[:, None] + 1e-16) + skip_ref[...],
                               0.0)


def _finish(acc2, den2, skip):
    R = 2048
    return pl.pallas_call(
        _finish_body,
        grid=(NPAD // R,),
        in_specs=[
            pl.BlockSpec((NC, R, D), lambda i: (0, i, 0)),
            pl.BlockSpec((NC, R), lambda i: (0, i)),
            pl.BlockSpec((R, D), lambda i: (i, 0)),
        ],
        out_specs=pl.BlockSpec((R, D), lambda i: (i, 0)),
        out_shape=jax.ShapeDtypeStruct((NPAD, D), jnp.float32),
    )(acc2, den2, skip)


def kernel(x, edge_index, Wq, bq, Wk, bk, Wv, bv, Ws, bs):
    src = edge_index[0]
    dst = edge_index[1]
    q, k, v, skip = _proj(x, Wq, bq, Wk, bk, Wv, bv, Ws, bs)
    acc2, den2 = _edge_agg(q, k, v, src, dst)
    return _finish(acc2, den2, skip)[:N]


# speedup vs baseline: 11.0707x; 1.8657x over previous
import dataclasses
import functools

import jax
import jax.numpy as jnp
import numpy as np
from jax import lax
from jax.experimental import pallas as pl
from jax.experimental.pallas import tpu as pltpu
from jax.experimental.pallas import tpu_sc as plsc

N = 10000
E = 320000
D = 128
NC = 2
NS = 16
NW = NC * NS
EPW = E // NW
C = 32
EPW2 = 10240
NCHUNK = EPW2 // C
GW = 8
NGROUP = NCHUNK // GW
NSUPER = NCHUNK // 16
PK = 32
DROWS = 320
NPAD = DROWS * PK
ZCHUNKS = NPAD // C
DCHUNK = DROWS // NS


def _proj_body(x_ref, wq_ref, bq_ref, wk_ref, bk_ref, wv_ref, bv_ref,
               ws_ref, bs_ref, q_ref, k_ref, v_ref, s_ref):
    x = x_ref[...]
    dn = (((1,), (1,)), ((), ()))
    hi = lax.Precision.HIGHEST
    q_ref[...] = lax.dot_general(x, wq_ref[...], dn, precision=hi) + bq_ref[...]
    k_ref[...] = lax.dot_general(x, wk_ref[...], dn, precision=hi) + bk_ref[...]
    v_ref[...] = lax.dot_general(x, wv_ref[...], dn, precision=hi) + bv_ref[...]
    s_ref[...] = lax.dot_general(x, ws_ref[...], dn, precision=hi) + bs_ref[...]


def _proj(x, Wq, bq, Wk, bk, Wv, bv, Ws, bs):
    R = 2048
    row_spec = pl.BlockSpec((R, D), lambda i: (i, 0))
    w_spec = pl.BlockSpec((D, D), lambda i: (0, 0))
    b_spec = pl.BlockSpec((D,), lambda i: (0,))
    out = jax.ShapeDtypeStruct((NPAD, D), jnp.float32)
    return pl.pallas_call(
        _proj_body,
        grid=(NPAD // R,),
        in_specs=[row_spec] + [w_spec, b_spec] * 4,
        out_specs=[row_spec] * 4,
        out_shape=[out] * 4,
    )(x, Wq, bq, Wk, bk, Wv, bv, Ws, bs)


def _edge_agg(q, k, v, src, dst):
    mesh = plsc.VectorSubcoreMesh(core_axis_name="c", subcore_axis_name="s")
    inv_sqrt_d = np.float32(1.0 / np.sqrt(D))
    cp = pltpu.CompilerParams()
    if "needs_layout_passes" in pltpu.CompilerParams.__dataclass_fields__:
        cp = dataclasses.replace(cp, needs_layout_passes=False)

    @functools.partial(
        pl.kernel,
        out_type=(
            jax.ShapeDtypeStruct((NC, NPAD, D), jnp.float32),
            jax.ShapeDtypeStruct((NC, NPAD), jnp.float32),
        ),
        mesh=mesh,
        compiler_params=cp,
        scratch_types=[
            pltpu.VMEM((2, GW, C), jnp.int32),
            pltpu.VMEM((2, GW, C), jnp.int32),
            pltpu.VMEM((2, C), jnp.int32),
            pltpu.VMEM((2, C, D), jnp.float32),
            pltpu.VMEM((2, C, D), jnp.float32),
            pltpu.VMEM((2, C, D), jnp.float32),
            pltpu.VMEM((2, C, D), jnp.float32),
            pltpu.VMEM((2, C, D), jnp.float32),
            pltpu.VMEM((C,), jnp.float32),
            pltpu.VMEM((DCHUNK * PK,), jnp.float32),
            pltpu.SemaphoreType.DMA,
            pltpu.SemaphoreType.DMA,
            pltpu.SemaphoreType.DMA,
            pltpu.SemaphoreType.DMA,
            pltpu.SemaphoreType.DMA,
            pltpu.SemaphoreType.DMA,
            pltpu.VMEM_SHARED((NPAD, D), jnp.float32),
            pltpu.VMEM_SHARED((DROWS, D), jnp.float32),
        ],
    )
    def agg_kernel(q_hbm, k_hbm, v_hbm, src_hbm, dst_hbm, agg_hbm, den_hbm,
                   sidxg, didxg, didxp2, qg2, kg2, vg2, msg2, denmsg2,
                   abuf, den1, gsem0, gsem1, ssem0, ssem1, isem0, isem1,
                   acc, dent):
        gsem = [gsem0, gsem1]
        ssem = [ssem0, ssem1]
        isem = [isem0, isem1]
        cid = lax.axis_index("c")
        sid = lax.axis_index("s")
        wid = cid * NS + sid
        zv = jnp.zeros((16,), jnp.float32)
        lane = jnp.arange(16, dtype=jnp.int32)
        mask15 = lane == 15

        @pl.loop(0, C)
        def _(r):
            for j in range(D // 16):
                msg2[0, r, pl.ds(j * 16, 16)] = zv
                denmsg2[0, r, pl.ds(j * 16, 16)] = zv
                denmsg2[1, r, pl.ds(j * 16, 16)] = zv

        zsrc = msg2.at[0]

        @pl.loop(0, ZCHUNKS // NS)
        def _(t):
            rc = sid + t * NS
            pltpu.sync_copy(zsrc, acc.at[pl.ds(rc * C, C)])

        pltpu.sync_copy(zsrc.at[pl.ds(0, DCHUNK)],
                        dent.at[pl.ds(sid * DCHUNK, DCHUNK)])

        pltpu.sync_copy(src_hbm.at[wid, pl.ds(0, GW)], sidxg.at[0])
        pltpu.sync_copy(dst_hbm.at[wid, pl.ds(0, GW)], didxg.at[0])
        pltpu.sync_copy(src_hbm.at[wid, pl.ds(GW, GW)], sidxg.at[1])
        pltpu.sync_copy(dst_hbm.at[wid, pl.ds(GW, GW)], didxg.at[1])
        plsc.subcore_barrier()

        def fetch(b, gslot, row):
            pltpu.make_async_copy(q_hbm.at[didxg.at[gslot, row]], qg2.at[b],
                                  gsem[b]).start()
            pltpu.make_async_copy(k_hbm.at[sidxg.at[gslot, row]], kg2.at[b],
                                  gsem[b]).start()
            pltpu.make_async_copy(v_hbm.at[sidxg.at[gslot, row]], vg2.at[b],
                                  gsem[b]).start()

        def wait_fetch(b, gslot, row):
            pltpu.make_async_copy(q_hbm.at[didxg.at[gslot, row]], qg2.at[b],
                                  gsem[b]).wait()
            pltpu.make_async_copy(k_hbm.at[sidxg.at[gslot, row]], kg2.at[b],
                                  gsem[b]).wait()
            pltpu.make_async_copy(v_hbm.at[sidxg.at[gslot, row]], vg2.at[b],
                                  gsem[b]).wait()

        def issue_scatter(b, gslot, row):
            pltpu.async_copy(msg2.at[b], acc.at[didxg.at[gslot, row]],
                             ssem[b], add=True)
            pltpu.async_copy(denmsg2.at[b], dent.at[didxp2.at[b]], ssem[b],
                             add=True)

        def drain_scatter(b, gslot, row):
            pltpu.make_async_copy(msg2.at[b], acc.at[didxg.at[gslot, row]],
                                  ssem[b]).wait()
            pltpu.make_async_copy(denmsg2.at[b], dent.at[didxp2.at[b]],
                                  ssem[b]).wait()
            for e0v in range(0, C, 16):
                dv = didxg[gslot, row, pl.ds(e0v, 16)]
                cols = lax.shift_left(lax.bitwise_and(dv, 31), 2)
                plsc.store_scatter(denmsg2.at[b], [lane + e0v, cols], zv)

        def prefetch_idx(g, islot):
            pltpu.make_async_copy(src_hbm.at[wid, pl.ds(g * GW, GW)],
                                  sidxg.at[islot], isem[islot]).start()
            pltpu.make_async_copy(dst_hbm.at[wid, pl.ds(g * GW, GW)],
                                  didxg.at[islot], isem[islot]).start()

        def wait_idx(g, islot):
            pltpu.make_async_copy(src_hbm.at[wid, pl.ds(g * GW, GW)],
                                  sidxg.at[islot], isem[islot]).wait()
            pltpu.make_async_copy(dst_hbm.at[wid, pl.ds(g * GW, GW)],
                                  didxg.at[islot], isem[islot]).wait()

        def compute(b, gslot, row):
            @pl.loop(0, C, step=4)
            def _(e0):
                for u in range(4):
                    e = e0 + u
                    p = [qg2[b, e, pl.ds(16 * j, 16)] *
                         kg2[b, e, pl.ds(16 * j, 16)] for j in range(8)]
                    t = ((p[0] + p[1]) + (p[2] + p[3])) + \
                        ((p[4] + p[5]) + (p[6] + p[7]))
                    c = plsc.cumsum(t)
                    wv = jnp.exp(c * inv_sqrt_d)
                    plsc.store_scatter(abuf, [jnp.full((16,), e, jnp.int32)],
                                       wv, mask=mask15)

            for e0v in range(0, C, 16):
                wv16 = abuf[pl.ds(e0v, 16)]
                dv = didxg[gslot, row, pl.ds(e0v, 16)]
                didxp2[b, pl.ds(e0v, 16)] = lax.shift_right_logical(dv, 5)
                cols = lax.shift_left(lax.bitwise_and(dv, 31), 2)
                plsc.store_scatter(denmsg2.at[b], [lane + e0v, cols], wv16)

            @pl.loop(0, C // 16)
            def _(g):
                e0g = g * 16
                wv16 = abuf[pl.ds(e0g, 16)]
                for u in range(16):
                    w = wv16[u]
                    for j in range(8):
                        msg2[b, e0g + u, pl.ds(16 * j, 16)] = \
                            vg2[b, e0g + u, pl.ds(16 * j, 16)] * w

        fetch(0, 0, 0)

        @pl.loop(0, NSUPER)
        def _(t):
            i0 = 16 * t
            for j in range(16):
                i = i0 + j
                b = j % 2
                gslot = (j // 8) % 2
                row = j % 8
                ngslot = ((j + 1) // 8) % 2
                nrow = (j + 1) % 8

                @pl.when(i + 1 < NCHUNK)
                def _():
                    if j == 15:
                        wait_idx(2 * t + 2, 0)
                    if j == 7:
                        @pl.when(t >= 1)
                        def _():
                            wait_idx(2 * t + 1, 1)
                    fetch(1 - b, ngslot, nrow)

                wait_fetch(b, gslot, row)

                pgslot = ((j - 2) // 8) % 2 if j >= 2 else 1
                prow = (j - 2) % 8

                @pl.when(i >= 2)
                def _():
                    drain_scatter(b, pgslot, prow)

                if j == 1:
                    @pl.when(t >= 1)
                    def _():
                        prefetch_idx(2 * t + 1, 1)

                if j == 9:
                    @pl.when(2 * t + 2 < NGROUP)
                    def _():
                        prefetch_idx(2 * t + 2, 0)

                compute(b, gslot, row)
                issue_scatter(b, gslot, row)

        drain_scatter(0, 1, 6)
        drain_scatter(1, 1, 7)
        plsc.subcore_barrier()

        @pl.loop(0, ZCHUNKS // NS)
        def _(t):
            rc = sid + t * NS
            pltpu.sync_copy(acc.at[pl.ds(rc * C, C)],
                            agg_hbm.at[cid, pl.ds(rc * C, C)])

        stage = denmsg2.at[0]
        pltpu.sync_copy(dent.at[pl.ds(sid * DCHUNK, DCHUNK)],
                        stage.at[pl.ds(0, DCHUNK)])

        @pl.loop(0, DCHUNK * PK // 16)
        def _(g):
            n = g * 16 + lane
            rows = lax.shift_right_logical(n, 5)
            cols = lax.shift_left(lax.bitwise_and(n, 31), 2)
            den1[pl.ds(g * 16, 16)] = plsc.load_gather(stage, [rows, cols])

        pltpu.sync_copy(den1, den_hbm.at[cid, pl.ds(sid * DCHUNK * PK,
                                                    DCHUNK * PK)])

    return agg_kernel(q, k, v, src, dst)


def _finish_body(acc_ref, den_ref, skip_ref, out_ref):
    agg = acc_ref[0] + acc_ref[1]
    den = den_ref[0] + den_ref[1]
    out_ref[...] = jnp.maximum(agg / (den[:, None] + 1e-16) + skip_ref[...],
                               0.0)


def _finish(acc2, den2, skip):
    R = 2048
    return pl.pallas_call(
        _finish_body,
        grid=(NPAD // R,),
        in_specs=[
            pl.BlockSpec((NC, R, D), lambda i: (0, i, 0)),
            pl.BlockSpec((NC, R), lambda i: (0, i)),
            pl.BlockSpec((R, D), lambda i: (i, 0)),
        ],
        out_specs=pl.BlockSpec((R, D), lambda i: (i, 0)),
        out_shape=jax.ShapeDtypeStruct((NPAD, D), jnp.float32),
    )(acc2, den2, skip)


def kernel(x, edge_index, Wq, bq, Wk, bk, Wv, bv, Ws, bs):
    xp = jnp.concatenate([x, jnp.zeros((NPAD - N, D), x.dtype)], axis=0)
    pad = EPW2 - EPW
    pad_rows = N + (jnp.arange(NW * pad, dtype=jnp.int32)
                    % (NPAD - N)).reshape(NW, pad)
    src2 = edge_index[0].reshape(NW, EPW)
    dst2 = edge_index[1].reshape(NW, EPW)
    srcp = jnp.concatenate([src2, pad_rows], axis=1).reshape(NW, NCHUNK, C)
    dstp = jnp.concatenate([dst2, pad_rows], axis=1).reshape(NW, NCHUNK, C)
    q, k, v, skip = _proj(xp, Wq, bq, Wk, bk, Wv, bv, Ws, bs)
    acc2, den2 = _edge_agg(q, k, v, srcp, dstp)
    return _finish(acc2, den2, skip)[:N]


# speedup vs baseline: 11.3043x; 1.0211x over previous
import dataclasses
import functools

import jax
import jax.numpy as jnp
import numpy as np
from jax import lax
from jax.experimental import pallas as pl
from jax.experimental.pallas import tpu as pltpu
from jax.experimental.pallas import tpu_sc as plsc

N = 10000
E = 320000
D = 128
NC = 2
NS = 16
NW = NC * NS
EPW = E // NW
C = 32
EPW2 = 10240
NCHUNK = EPW2 // C
GW = 8
NGROUP = NCHUNK // GW
NSUPER = NCHUNK // 16
PK = 32
DROWS = 320
NPAD = DROWS * PK
ZCHUNKS = NPAD // C
DCHUNK = DROWS // NS


def _proj_body(x_ref, wq_ref, bq_ref, wk_ref, bk_ref, wv_ref, bv_ref,
               ws_ref, bs_ref, q_ref, k_ref, v_ref, s_ref):
    x = x_ref[...]
    dn = (((1,), (1,)), ((), ()))
    hi = lax.Precision.HIGHEST
    q_ref[...] = lax.dot_general(x, wq_ref[...], dn, precision=hi) + bq_ref[...]
    k_ref[...] = lax.dot_general(x, wk_ref[...], dn, precision=hi) + bk_ref[...]
    v_ref[...] = lax.dot_general(x, wv_ref[...], dn, precision=hi) + bv_ref[...]
    s_ref[...] = lax.dot_general(x, ws_ref[...], dn, precision=hi) + bs_ref[...]


def _proj(x, Wq, bq, Wk, bk, Wv, bv, Ws, bs):
    R = 2048
    row_spec = pl.BlockSpec((R, D), lambda i: (i, 0))
    w_spec = pl.BlockSpec((D, D), lambda i: (0, 0))
    b_spec = pl.BlockSpec((D,), lambda i: (0,))
    out = jax.ShapeDtypeStruct((NPAD, D), jnp.float32)
    return pl.pallas_call(
        _proj_body,
        grid=(NPAD // R,),
        in_specs=[row_spec] + [w_spec, b_spec] * 4,
        out_specs=[row_spec] * 4,
        out_shape=[out] * 4,
    )(x, Wq, bq, Wk, bk, Wv, bv, Ws, bs)


def _edge_agg(q, k, v, src, dst):
    mesh = plsc.VectorSubcoreMesh(core_axis_name="c", subcore_axis_name="s")
    inv_sqrt_d = np.float32(1.0 / np.sqrt(D))
    cp = pltpu.CompilerParams()
    if "needs_layout_passes" in pltpu.CompilerParams.__dataclass_fields__:
        cp = dataclasses.replace(cp, needs_layout_passes=False)

    @functools.partial(
        pl.kernel,
        out_type=(
            jax.ShapeDtypeStruct((NC, NPAD, D), jnp.float32),
            jax.ShapeDtypeStruct((NC, NPAD), jnp.float32),
        ),
        mesh=mesh,
        compiler_params=cp,
        scratch_types=[
            pltpu.VMEM((2, GW, C), jnp.int32),
            pltpu.VMEM((2, GW, C), jnp.int32),
            pltpu.VMEM((2, C), jnp.int32),
            pltpu.VMEM((2, C, D), jnp.float32),
            pltpu.VMEM((2, C, D), jnp.float32),
            pltpu.VMEM((2, C, D), jnp.float32),
            pltpu.VMEM((2, C, D), jnp.float32),
            pltpu.VMEM((2, C, D), jnp.float32),
            pltpu.VMEM((C,), jnp.float32),
            pltpu.VMEM((DCHUNK * PK,), jnp.float32),
            pltpu.SemaphoreType.DMA,
            pltpu.SemaphoreType.DMA,
            pltpu.SemaphoreType.DMA,
            pltpu.SemaphoreType.DMA,
            pltpu.SemaphoreType.DMA,
            pltpu.SemaphoreType.DMA,
            pltpu.VMEM_SHARED((NPAD, D), jnp.float32),
            pltpu.VMEM_SHARED((DROWS, D), jnp.float32),
        ],
    )
    def agg_kernel(q_hbm, k_hbm, v_hbm, src_hbm, dst_hbm, agg_hbm, den_hbm,
                   sidxg, didxg, didxp2, qg2, kg2, vg2, msg2, denmsg2,
                   abuf, den1, gsem0, gsem1, ssem0, ssem1, isem0, isem1,
                   acc, dent):
        gsem = [gsem0, gsem1]
        ssem = [ssem0, ssem1]
        isem = [isem0, isem1]
        cid = lax.axis_index("c")
        sid = lax.axis_index("s")
        wid = cid * NS + sid
        zv = jnp.zeros((16,), jnp.float32)
        lane = jnp.arange(16, dtype=jnp.int32)
        mask15 = lane == 15

        @pl.loop(0, C)
        def _(r):
            for j in range(D // 16):
                msg2[0, r, pl.ds(j * 16, 16)] = zv
                denmsg2[0, r, pl.ds(j * 16, 16)] = zv
                denmsg2[1, r, pl.ds(j * 16, 16)] = zv

        zsrc = msg2.at[0]

        @pl.loop(0, ZCHUNKS // NS)
        def _(t):
            rc = sid + t * NS
            pltpu.make_async_copy(zsrc, acc.at[pl.ds(rc * C, C)],
                                  gsem0).start()

        pltpu.make_async_copy(zsrc.at[pl.ds(0, DCHUNK)],
                              dent.at[pl.ds(sid * DCHUNK, DCHUNK)],
                              gsem0).start()

        @pl.loop(0, ZCHUNKS // NS)
        def _(t):
            rc = sid + t * NS
            pltpu.make_async_copy(zsrc, acc.at[pl.ds(rc * C, C)],
                                  gsem0).wait()

        pltpu.make_async_copy(zsrc.at[pl.ds(0, DCHUNK)],
                              dent.at[pl.ds(sid * DCHUNK, DCHUNK)],
                              gsem0).wait()

        pltpu.sync_copy(src_hbm.at[wid, pl.ds(0, GW)], sidxg.at[0])
        pltpu.sync_copy(dst_hbm.at[wid, pl.ds(0, GW)], didxg.at[0])
        pltpu.sync_copy(src_hbm.at[wid, pl.ds(GW, GW)], sidxg.at[1])
        pltpu.sync_copy(dst_hbm.at[wid, pl.ds(GW, GW)], didxg.at[1])
        plsc.subcore_barrier()

        def fetch(b, gslot, row):
            pltpu.make_async_copy(q_hbm.at[didxg.at[gslot, row]], qg2.at[b],
                                  gsem[b]).start()
            pltpu.make_async_copy(k_hbm.at[sidxg.at[gslot, row]], kg2.at[b],
                                  gsem[b]).start()
            pltpu.make_async_copy(v_hbm.at[sidxg.at[gslot, row]], vg2.at[b],
                                  gsem[b]).start()

        def wait_fetch(b, gslot, row):
            pltpu.make_async_copy(q_hbm.at[didxg.at[gslot, row]], qg2.at[b],
                                  gsem[b]).wait()
            pltpu.make_async_copy(k_hbm.at[sidxg.at[gslot, row]], kg2.at[b],
                                  gsem[b]).wait()
            pltpu.make_async_copy(v_hbm.at[sidxg.at[gslot, row]], vg2.at[b],
                                  gsem[b]).wait()

        def issue_scatter(b, gslot, row):
            pltpu.async_copy(msg2.at[b], acc.at[didxg.at[gslot, row]],
                             ssem[b], add=True)
            pltpu.async_copy(denmsg2.at[b], dent.at[didxp2.at[b]], ssem[b],
                             add=True)

        def drain_scatter(b, gslot, row):
            pltpu.make_async_copy(msg2.at[b], acc.at[didxg.at[gslot, row]],
                                  ssem[b]).wait()
            pltpu.make_async_copy(denmsg2.at[b], dent.at[didxp2.at[b]],
                                  ssem[b]).wait()
            for e0v in range(0, C, 16):
                dv = didxg[gslot, row, pl.ds(e0v, 16)]
                cols = lax.shift_left(lax.bitwise_and(dv, 31), 2)
                plsc.store_scatter(denmsg2.at[b], [lane + e0v, cols], zv)

        def prefetch_idx(g, islot):
            pltpu.make_async_copy(src_hbm.at[wid, pl.ds(g * GW, GW)],
                                  sidxg.at[islot], isem[islot]).start()
            pltpu.make_async_copy(dst_hbm.at[wid, pl.ds(g * GW, GW)],
                                  didxg.at[islot], isem[islot]).start()

        def wait_idx(g, islot):
            pltpu.make_async_copy(src_hbm.at[wid, pl.ds(g * GW, GW)],
                                  sidxg.at[islot], isem[islot]).wait()
            pltpu.make_async_copy(dst_hbm.at[wid, pl.ds(g * GW, GW)],
                                  didxg.at[islot], isem[islot]).wait()

        def compute(b, gslot, row):
            @pl.loop(0, C, step=4)
            def _(e0):
                for u in range(4):
                    e = e0 + u
                    p = [qg2[b, e, pl.ds(16 * j, 16)] *
                         kg2[b, e, pl.ds(16 * j, 16)] for j in range(8)]
                    t = ((p[0] + p[1]) + (p[2] + p[3])) + \
                        ((p[4] + p[5]) + (p[6] + p[7]))
                    c = plsc.cumsum(t)
                    wv = jnp.exp(c * inv_sqrt_d)
                    plsc.store_scatter(abuf, [jnp.full((16,), e, jnp.int32)],
                                       wv, mask=mask15)

            for e0v in range(0, C, 16):
                wv16 = abuf[pl.ds(e0v, 16)]
                dv = didxg[gslot, row, pl.ds(e0v, 16)]
                didxp2[b, pl.ds(e0v, 16)] = lax.shift_right_logical(dv, 5)
                cols = lax.shift_left(lax.bitwise_and(dv, 31), 2)
                plsc.store_scatter(denmsg2.at[b], [lane + e0v, cols], wv16)

            @pl.loop(0, C // 16)
            def _(g):
                e0g = g * 16
                wv16 = abuf[pl.ds(e0g, 16)]
                for u in range(16):
                    w = wv16[u]
                    for j in range(8):
                        msg2[b, e0g + u, pl.ds(16 * j, 16)] = \
                            vg2[b, e0g + u, pl.ds(16 * j, 16)] * w

        fetch(0, 0, 0)

        @pl.loop(0, NSUPER)
        def _(t):
            i0 = 16 * t
            for j in range(16):
                i = i0 + j
                b = j % 2
                gslot = (j // 8) % 2
                row = j % 8
                ngslot = ((j + 1) // 8) % 2
                nrow = (j + 1) % 8

                @pl.when(i + 1 < NCHUNK)
                def _():
                    if j == 15:
                        wait_idx(2 * t + 2, 0)
                    if j == 7:
                        @pl.when(t >= 1)
                        def _():
                            wait_idx(2 * t + 1, 1)
                    fetch(1 - b, ngslot, nrow)

                wait_fetch(b, gslot, row)

                pgslot = ((j - 2) // 8) % 2 if j >= 2 else 1
                prow = (j - 2) % 8

                @pl.when(i >= 2)
                def _():
                    drain_scatter(b, pgslot, prow)

                if j == 1:
                    @pl.when(t >= 1)
                    def _():
                        prefetch_idx(2 * t + 1, 1)

                if j == 9:
                    @pl.when(2 * t + 2 < NGROUP)
                    def _():
                        prefetch_idx(2 * t + 2, 0)

                compute(b, gslot, row)
                issue_scatter(b, gslot, row)

        drain_scatter(0, 1, 6)
        drain_scatter(1, 1, 7)
        plsc.subcore_barrier()

        @pl.loop(0, ZCHUNKS // NS)
        def _(t):
            rc = sid + t * NS
            pltpu.make_async_copy(acc.at[pl.ds(rc * C, C)],
                                  agg_hbm.at[cid, pl.ds(rc * C, C)],
                                  gsem0).start()

        @pl.loop(0, ZCHUNKS // NS)
        def _(t):
            rc = sid + t * NS
            pltpu.make_async_copy(acc.at[pl.ds(rc * C, C)],
                                  agg_hbm.at[cid, pl.ds(rc * C, C)],
                                  gsem0).wait()

        stage = denmsg2.at[0]
        pltpu.sync_copy(dent.at[pl.ds(sid * DCHUNK, DCHUNK)],
                        stage.at[pl.ds(0, DCHUNK)])

        @pl.loop(0, DCHUNK * PK // 16)
        def _(g):
            n = g * 16 + lane
            rows = lax.shift_right_logical(n, 5)
            cols = lax.shift_left(lax.bitwise_and(n, 31), 2)
            den1[pl.ds(g * 16, 16)] = plsc.load_gather(stage, [rows, cols])

        pltpu.sync_copy(den1, den_hbm.at[cid, pl.ds(sid * DCHUNK * PK,
                                                    DCHUNK * PK)])

    return agg_kernel(q, k, v, src, dst)


def _finish_body(acc_ref, den_ref, skip_ref, out_ref):
    agg = acc_ref[0] + acc_ref[1]
    den = den_ref[0] + den_ref[1]
    out_ref[...] = jnp.maximum(agg / (den[:, None] + 1e-16) + skip_ref[...],
                               0.0)


def _finish(acc2, den2, skip):
    R = 2048
    return pl.pallas_call(
        _finish_body,
        grid=(NPAD // R,),
        in_specs=[
            pl.BlockSpec((NC, R, D), lambda i: (0, i, 0)),
            pl.BlockSpec((NC, R), lambda i: (0, i)),
            pl.BlockSpec((R, D), lambda i: (i, 0)),
        ],
        out_specs=pl.BlockSpec((R, D), lambda i: (i, 0)),
        out_shape=jax.ShapeDtypeStruct((NPAD, D), jnp.float32),
    )(acc2, den2, skip)


def kernel(x, edge_index, Wq, bq, Wk, bk, Wv, bv, Ws, bs):
    xp = jnp.concatenate([x, jnp.zeros((NPAD - N, D), x.dtype)], axis=0)
    pad = EPW2 - EPW
    pad_rows = N + (jnp.arange(NW * pad, dtype=jnp.int32)
                    % (NPAD - N)).reshape(NW, pad)
    src2 = edge_index[0].reshape(NW, EPW)
    dst2 = edge_index[1].reshape(NW, EPW)
    srcp = jnp.concatenate([src2, pad_rows], axis=1).reshape(NW, NCHUNK, C)
    dstp = jnp.concatenate([dst2, pad_rows], axis=1).reshape(NW, NCHUNK, C)
    q, k, v, skip = _proj(xp, Wq, bq, Wk, bk, Wv, bv, Ws, bs)
    acc2, den2 = _edge_agg(q, k, v, srcp, dstp)
    return _finish(acc2, den2, skip)[:N]
